# Initial kernel scaffold; baseline (speedup 1.0000x reference)
import jax
import jax.numpy as jnp
from jax.experimental import pallas as pl


def kernel(x, edge_index, node_time, edge_time, time_w, time_b, lin_w, lin_b, key_w, key_b, query_w, query_b, value_w, value_b, edge_w, skip_w, skip_b, out_w, out_b):
    raise NotImplementedError("write your pallas kernel here")


# speedup vs baseline: 12.0838x; 12.0838x over previous
import functools

import jax
import jax.numpy as jnp
from jax import lax
from jax.experimental import pallas as pl
from jax.experimental.pallas import tpu as pltpu
from jax.experimental.pallas import tpu_sc as plsc

N = 10000
E = 320000
D_IN = 128
HC = 32
NCLS = 2

NC = 2
NS = 16
NW = NC * NS
L = 16

B = 128
BLKS = -(-E // (NW * B))
EPW = BLKS * B
E_PAD = NW * EPW

ROWS_PER_TILE = 640
NP = NS * ROWS_PER_TILE
QROWS = N + L


def _tc_proj_body(x_ref, lwT, lb, qwT, qb, kwT, kb, vwT, vb, swT, sb,
                  qn, kn, vn, hs):
    xb = x_ref[...]
    h1 = jnp.maximum(jnp.dot(xb, lwT[...], preferred_element_type=jnp.float32)
                     + lb[...], 0.0)
    qn[...] = jnp.dot(h1, qwT[...], preferred_element_type=jnp.float32) + qb[...]
    kn[...] = jnp.dot(h1, kwT[...], preferred_element_type=jnp.float32) + kb[...]
    vn[...] = jnp.dot(h1, vwT[...], preferred_element_type=jnp.float32) + vb[...]
    hs[...] = jnp.dot(h1, swT[...], preferred_element_type=jnp.float32) + sb[...]


def _tc_proj(x, lin_w, lin_b, key_w, key_b, query_w, query_b, value_w, value_b,
             skip_w, skip_b):
    R = 2000
    grid = (N // R,)
    row_spec = pl.BlockSpec((R, D_IN), lambda i: (i, 0))
    out_spec = pl.BlockSpec((R, HC), lambda i: (i, 0))
    full = lambda shape: pl.BlockSpec(shape, lambda i: (0, 0))
    return pl.pallas_call(
        _tc_proj_body,
        grid=grid,
        in_specs=[row_spec,
                  full((D_IN, HC)), full((1, HC)),
                  full((HC, HC)), full((1, HC)),
                  full((HC, HC)), full((1, HC)),
                  full((HC, HC)), full((1, HC)),
                  full((HC, HC)), full((1, HC))],
        out_specs=[out_spec] * 4,
        out_shape=[jax.ShapeDtypeStruct((N, HC), jnp.float32)] * 4,
    )(x, lin_w.T, lin_b.reshape(1, HC),
      query_w.T, query_b.reshape(1, HC),
      key_w.T, key_b.reshape(1, HC),
      value_w.T, value_b.reshape(1, HC),
      skip_w.T, skip_b.reshape(1, HC))


def _sc_relt_body(nt_hbm, src_hbm, et_hbm, rel_hbm, nt_v, src_v, et_v, rel_v):
    cid = lax.axis_index("c")
    sid = lax.axis_index("s")
    wid = cid * NS + sid
    wbase = wid * EPW
    pltpu.sync_copy(nt_hbm, nt_v)
    iota = lax.iota(jnp.int32, L)

    def blk_body(blk, _):
        base = wbase + blk * B
        pltpu.sync_copy(src_hbm.at[pl.ds(base, B)], src_v)
        pltpu.sync_copy(et_hbm.at[pl.ds(base, B)], et_v)

        def grp_body(g, _):
            off = g * L
            sidx = src_v[pl.ds(off, L)]
            r = plsc.load_gather(nt_v, [sidx]) - et_v[pl.ds(off, L)]
            rel_v[pl.ds(off, L)] = r
            return 0

        lax.fori_loop(0, B // L, grp_body, 0)
        pltpu.sync_copy(rel_v, rel_hbm.at[pl.ds(base, B)])
        return 0

    lax.fori_loop(0, BLKS, blk_body, 0)


def _sc_relt(node_time, src_pad, et_pad):
    mesh = plsc.VectorSubcoreMesh(core_axis_name="c", subcore_axis_name="s")
    return pl.kernel(
        _sc_relt_body,
        out_type=jax.ShapeDtypeStruct((E_PAD,), jnp.float32),
        mesh=mesh,
        scratch_types=[
            pltpu.VMEM((N,), jnp.float32),
            pltpu.VMEM((B,), jnp.int32),
            pltpu.VMEM((B,), jnp.float32),
            pltpu.VMEM((B,), jnp.float32),
        ],
        compiler_params=pltpu.CompilerParams(needs_layout_passes=False, use_tc_tiling_on_sc=False),
    )(node_time, src_pad, et_pad)


def _tc_emat_body(rel_ref, twr, tbr, ewT, out_ref):
    r = rel_ref[0]
    enc = jnp.cos(r * twr[...] + tbr[...])
    out_ref[0] = jnp.dot(enc, ewT[...], preferred_element_type=jnp.float32)


def _tc_emat(rel_pad, time_w, time_b, edge_w):
    RB = 1024
    grid = (E_PAD // RB,)
    rel3 = rel_pad.reshape(E_PAD // RB, RB, 1)
    out = pl.pallas_call(
        _tc_emat_body,
        grid=grid,
        in_specs=[pl.BlockSpec((1, RB, 1), lambda i: (i, 0, 0)),
                  pl.BlockSpec((1, HC), lambda i: (0, 0)),
                  pl.BlockSpec((1, HC), lambda i: (0, 0)),
                  pl.BlockSpec((HC, HC), lambda i: (0, 0))],
        out_specs=pl.BlockSpec((1, RB, HC), lambda i: (i, 0, 0)),
        out_shape=jax.ShapeDtypeStruct((E_PAD // RB, RB, HC), jnp.float32),
    )(rel3, time_w.reshape(1, HC), time_b.reshape(1, HC), edge_w.T)
    return out.reshape(E_PAD, HC)


def _sc_attn_body(qn_hbm, kn_hbm, vn_hbm, em_hbm, src_hbm, dst_hbm, acc_hbm,
                  src_v, dst_v, qrows, krows, vrows, erows, zmsg, msg, acc_sh):
    cid = lax.axis_index("c")
    sid = lax.axis_index("s")
    wid = cid * NS + sid
    wbase = wid * EPW
    iota = lax.iota(jnp.int32, L)
    zf = jnp.zeros((L,), jnp.float32)

    def zrow(i, _):
        for j in range(3):
            zmsg[i, pl.ds(j * L, L)] = zf
        return 0
    lax.fori_loop(0, ROWS_PER_TILE, zrow, 0)
    tbase = sid * ROWS_PER_TILE
    pltpu.sync_copy(zmsg, acc_sh.at[pl.ds(tbase, ROWS_PER_TILE)])
    plsc.subcore_barrier()

    def mrow(i, _):
        msg[i, pl.ds(2 * L, L)] = zf
        return 0
    lax.fori_loop(0, B, mrow, 0)

    def blk_body(blk, _):
        base = wbase + blk * B
        pltpu.sync_copy(src_hbm.at[pl.ds(base, B)], src_v)
        pltpu.sync_copy(dst_hbm.at[pl.ds(base, B)], dst_v)
        pltpu.sync_copy(qn_hbm.at[dst_v], qrows)
        pltpu.sync_copy(kn_hbm.at[src_v], krows)
        pltpu.sync_copy(vn_hbm.at[src_v], vrows)
        pltpu.sync_copy(em_hbm.at[pl.ds(base, B)], erows)

        def grp_body(g, _):
            ids = g * L + iota
            a0 = zf
            a1 = zf
            for c in range(HC):
                cv = jnp.full((L,), c, jnp.int32)
                qc = plsc.load_gather(qrows, [ids, cv])
                kc = plsc.load_gather(krows, [ids, cv])
                ec = plsc.load_gather(erows, [ids, cv])
                t = qc * (kc + ec)
                if c < 16:
                    a0 = a0 + t
                else:
                    a1 = a1 + t
            p0 = jnp.exp(a0 * 0.25)
            p1 = jnp.exp(a1 * 0.25)
            for c in range(HC):
                cv = jnp.full((L,), c, jnp.int32)
                vc = plsc.load_gather(vrows, [ids, cv])
                ec = plsc.load_gather(erows, [ids, cv])
                p = p0 if c < 16 else p1
                plsc.store_scatter(msg, [ids, cv], p * (vc + ec))
            plsc.store_scatter(msg, [ids, jnp.full((L,), 32, jnp.int32)], p0)
            plsc.store_scatter(msg, [ids, jnp.full((L,), 33, jnp.int32)], p1)
            return 0

        lax.fori_loop(0, B // L, grp_body, 0)
        pltpu.sync_copy(msg, acc_sh.at[dst_v], add=True)
        return 0

    lax.fori_loop(0, BLKS, blk_body, 0)
    plsc.subcore_barrier()
    pltpu.sync_copy(acc_sh.at[pl.ds(tbase, ROWS_PER_TILE)],
                    acc_hbm.at[cid, pl.ds(tbase, ROWS_PER_TILE)])


def _sc_attn(qn_pad, kn, vn, e_mat, src_pad, dst_pad):
    mesh = plsc.VectorSubcoreMesh(core_axis_name="c", subcore_axis_name="s")
    return pl.kernel(
        _sc_attn_body,
        out_type=jax.ShapeDtypeStruct((NC, NP, 48), jnp.float32),
        mesh=mesh,
        scratch_types=[
            pltpu.VMEM((B,), jnp.int32),
            pltpu.VMEM((B,), jnp.int32),
            pltpu.VMEM((B, HC), jnp.float32),
            pltpu.VMEM((B, HC), jnp.float32),
            pltpu.VMEM((B, HC), jnp.float32),
            pltpu.VMEM((B, HC), jnp.float32),
            pltpu.VMEM((ROWS_PER_TILE, 48), jnp.float32),
            pltpu.VMEM((B, 48), jnp.float32),
            pltpu.VMEM_SHARED((NP, 48), jnp.float32),
        ],
        compiler_params=pltpu.CompilerParams(needs_layout_passes=False, use_tc_tiling_on_sc=False),
    )(qn_pad, kn, vn, e_mat, src_pad, dst_pad)


def _tc_final_body(a0_ref, a1_ref, hs_ref, owT, ob, out_ref):
    a = a0_ref[...] + a1_ref[...]
    d0 = a[:, 32:33] + 1e-16
    d1 = a[:, 33:34] + 1e-16
    h0 = a[:, 0:16] / d0
    h1 = a[:, 16:32] / d1
    w = owT[...]
    o = (jnp.dot(h0, w[0:16, :], preferred_element_type=jnp.float32)
         + jnp.dot(h1, w[16:32, :], preferred_element_type=jnp.float32)
         + jnp.dot(hs_ref[...], w, preferred_element_type=jnp.float32)
         + ob[...])
    m = jnp.max(o, axis=1, keepdims=True)
    s = o - m
    out_ref[...] = s - jnp.log(jnp.sum(jnp.exp(s), axis=1, keepdims=True))


def _tc_final(acc, hs, out_w, out_b):
    R = 2000
    grid = (N // R,)
    return pl.pallas_call(
        _tc_final_body,
        grid=grid,
        in_specs=[pl.BlockSpec((R, 48), lambda i: (i, 0)),
                  pl.BlockSpec((R, 48), lambda i: (i, 0)),
                  pl.BlockSpec((R, HC), lambda i: (i, 0)),
                  pl.BlockSpec((HC, NCLS), lambda i: (0, 0)),
                  pl.BlockSpec((1, NCLS), lambda i: (0, 0))],
        out_specs=pl.BlockSpec((R, NCLS), lambda i: (i, 0)),
        out_shape=jax.ShapeDtypeStruct((N, NCLS), jnp.float32),
    )(acc[0, :N], acc[1, :N], hs, out_w.T, out_b.reshape(1, NCLS))


def _sc_test(table, idx):
    V, D = table.shape
    BT = idx.shape[0]
    b_per_w = BT // NW
    mesh = plsc.VectorSubcoreMesh(core_axis_name="c", subcore_axis_name="s")

    @functools.partial(
        pl.kernel, mesh=mesh,
        out_type=jax.ShapeDtypeStruct((BT, D), jnp.float32),
        scratch_types=[
            pltpu.VMEM((b_per_w,), jnp.int32),
            pltpu.VMEM((b_per_w, D), jnp.float32),
            pltpu.SemaphoreType.DMA,
        ],
    )
    def k(table_hbm, idx_hbm, out_hbm, idx_v, rows_v, sem):
        wid = lax.axis_index("s") * NC + lax.axis_index("c")
        base = wid * b_per_w
        pltpu.sync_copy(idx_hbm.at[pl.ds(base, b_per_w)], idx_v)
        pltpu.async_copy(table_hbm.at[idx_v], rows_v, sem).wait()
        pltpu.sync_copy(rows_v, out_hbm.at[pl.ds(base, b_per_w)])

    return k(table, idx)


def _jnp_control(x, edge_index, node_time, edge_time, time_w, time_b, lin_w,
                 lin_b, key_w, key_b, query_w, query_b, value_w, value_b,
                 edge_w, skip_w, skip_b, out_w, out_b):
    src = edge_index[0]
    dst = edge_index[1]
    n = x.shape[0]
    rel_t = node_time[src][:, None] - edge_time
    rel_t_enc = jnp.cos(rel_t @ time_w.T + time_b)
    h1 = jax.nn.relu(x @ lin_w.T + lin_b)
    q = (h1 @ query_w.T + query_b).reshape(n, 2, 16)[dst]
    k = (h1 @ key_w.T + key_b).reshape(n, 2, 16)[src]
    v = (h1 @ value_w.T + value_b).reshape(n, 2, 16)[src]
    e = (rel_t_enc @ edge_w.T).reshape(-1, 2, 16)
    k = k + e
    v = v + e
    alpha = (q * k).sum(-1) * 0.25
    amax = jax.ops.segment_max(alpha, dst, num_segments=n)
    amax = jnp.where(jnp.isfinite(amax), amax, 0.0)
    alpha = jnp.exp(alpha - amax[dst])
    denom = jax.ops.segment_sum(alpha, dst, num_segments=n)
    alpha = alpha / (denom[dst] + 1e-16)
    msg = v * alpha[:, :, None]
    agg = jax.ops.segment_sum(msg, dst, num_segments=n)
    h = agg.reshape(n, 32) + (h1 @ skip_w.T + skip_b)
    out = h @ out_w.T + out_b
    return jax.nn.log_softmax(out, axis=1)


def _pipeline(x, edge_index, node_time, edge_time, time_w, time_b, lin_w,
              lin_b, key_w, key_b, query_w, query_b, value_w, value_b,
              edge_w, skip_w, skip_b, out_w, out_b):
    src = edge_index[0]
    dst = edge_index[1]
    pad = E_PAD - E
    src_pad = jnp.concatenate([src, jnp.zeros((pad,), jnp.int32)])
    dst_pad = jnp.concatenate([dst, jnp.full((pad,), N, jnp.int32)])
    et_pad = jnp.concatenate([edge_time.reshape(E), jnp.zeros((pad,), jnp.float32)])

    qn, kn, vn, hs = _tc_proj(x, lin_w, lin_b, key_w, key_b, query_w, query_b,
                              value_w, value_b, skip_w, skip_b)
    qn_pad = jnp.concatenate([qn, jnp.zeros((QROWS - N, HC), jnp.float32)])

    rel_pad = _sc_relt(node_time, src_pad, et_pad)
    e_mat = _tc_emat(rel_pad, time_w, time_b, edge_w)
    acc = _sc_attn(qn_pad, kn, vn, e_mat, src_pad, dst_pad)
    return _tc_final(acc, hs, out_w, out_b)


def kernel(x, edge_index, node_time, edge_time, time_w, time_b, lin_w, lin_b,
           key_w, key_b, query_w, query_b, value_w, value_b, edge_w,
           skip_w, skip_b, out_w, out_b):
    return _pipeline(x, edge_index, node_time, edge_time, time_w, time_b,
                     lin_w, lin_b, key_w, key_b, query_w, query_b, value_w,
                     value_b, edge_w, skip_w, skip_b, out_w, out_b)


# speedup vs baseline: 14.3806x; 1.1901x over previous
import functools

import jax
import jax.numpy as jnp
from jax import lax
from jax.experimental import pallas as pl
from jax.experimental.pallas import tpu as pltpu
from jax.experimental.pallas import tpu_sc as plsc

N = 10000
E = 320000
D_IN = 128
HC = 32
NCLS = 2

NC = 2
NS = 16
NW = NC * NS
L = 16

B = 128
BLKS = -(-E // (NW * B))
EPW = BLKS * B
E_PAD = NW * EPW

ROWS_PER_TILE = 640
NP = NS * ROWS_PER_TILE
QROWS = N + L


def _tc_proj_body(x_ref, lwT, lb, qwT, qb, kwT, kb, vwT, vb, swT, sb,
                  qn, kn, vn, hs):
    xb = x_ref[...]
    h1 = jnp.maximum(jnp.dot(xb, lwT[...], preferred_element_type=jnp.float32)
                     + lb[...], 0.0)
    qn[...] = jnp.dot(h1, qwT[...], preferred_element_type=jnp.float32) + qb[...]
    kn[...] = jnp.dot(h1, kwT[...], preferred_element_type=jnp.float32) + kb[...]
    vn[...] = jnp.dot(h1, vwT[...], preferred_element_type=jnp.float32) + vb[...]
    hs[...] = jnp.dot(h1, swT[...], preferred_element_type=jnp.float32) + sb[...]


def _tc_proj(x, lin_w, lin_b, key_w, key_b, query_w, query_b, value_w, value_b,
             skip_w, skip_b):
    R = 2000
    grid = (N // R,)
    row_spec = pl.BlockSpec((R, D_IN), lambda i: (i, 0))
    out_spec = pl.BlockSpec((R, HC), lambda i: (i, 0))
    full = lambda shape: pl.BlockSpec(shape, lambda i: (0, 0))
    return pl.pallas_call(
        _tc_proj_body,
        grid=grid,
        in_specs=[row_spec,
                  full((D_IN, HC)), full((1, HC)),
                  full((HC, HC)), full((1, HC)),
                  full((HC, HC)), full((1, HC)),
                  full((HC, HC)), full((1, HC)),
                  full((HC, HC)), full((1, HC))],
        out_specs=[out_spec] * 4,
        out_shape=[jax.ShapeDtypeStruct((N, HC), jnp.float32)] * 4,
    )(x, lin_w.T, lin_b.reshape(1, HC),
      query_w.T, query_b.reshape(1, HC),
      key_w.T, key_b.reshape(1, HC),
      value_w.T, value_b.reshape(1, HC),
      skip_w.T, skip_b.reshape(1, HC))


def _sc_relt_body(nt_hbm, src_hbm, et_hbm, rel_hbm, nt_v, src_v, et_v, rel_v):
    cid = lax.axis_index("c")
    sid = lax.axis_index("s")
    wid = cid * NS + sid
    wbase = wid * EPW
    pltpu.sync_copy(nt_hbm, nt_v)
    iota = lax.iota(jnp.int32, L)

    def blk_body(blk, _):
        base = wbase + blk * B
        pltpu.sync_copy(src_hbm.at[pl.ds(base, B)], src_v)
        pltpu.sync_copy(et_hbm.at[pl.ds(base, B)], et_v)

        def grp_body(g, _):
            off = g * L
            sidx = src_v[pl.ds(off, L)]
            r = plsc.load_gather(nt_v, [sidx]) - et_v[pl.ds(off, L)]
            rel_v[pl.ds(off, L)] = r
            return 0

        lax.fori_loop(0, B // L, grp_body, 0)
        pltpu.sync_copy(rel_v, rel_hbm.at[pl.ds(base, B)])
        return 0

    lax.fori_loop(0, BLKS, blk_body, 0)


def _sc_relt(node_time, src_pad, et_pad):
    mesh = plsc.VectorSubcoreMesh(core_axis_name="c", subcore_axis_name="s")
    return pl.kernel(
        _sc_relt_body,
        out_type=jax.ShapeDtypeStruct((E_PAD,), jnp.float32),
        mesh=mesh,
        scratch_types=[
            pltpu.VMEM((N,), jnp.float32),
            pltpu.VMEM((B,), jnp.int32),
            pltpu.VMEM((B,), jnp.float32),
            pltpu.VMEM((B,), jnp.float32),
        ],
        compiler_params=pltpu.CompilerParams(needs_layout_passes=False, use_tc_tiling_on_sc=False),
    )(node_time, src_pad, et_pad)


def _tc_emat_body(rel_ref, twr, tbr, ewT, out_ref):
    r = rel_ref[0]
    enc = jnp.cos(r * twr[...] + tbr[...])
    out_ref[0] = jnp.dot(enc, ewT[...], preferred_element_type=jnp.float32)


def _tc_emat(rel_pad, time_w, time_b, edge_w):
    RB = 1024
    grid = (E_PAD // RB,)
    rel3 = rel_pad.reshape(E_PAD // RB, RB, 1)
    out = pl.pallas_call(
        _tc_emat_body,
        grid=grid,
        in_specs=[pl.BlockSpec((1, RB, 1), lambda i: (i, 0, 0)),
                  pl.BlockSpec((1, HC), lambda i: (0, 0)),
                  pl.BlockSpec((1, HC), lambda i: (0, 0)),
                  pl.BlockSpec((HC, HC), lambda i: (0, 0))],
        out_specs=pl.BlockSpec((1, RB, HC), lambda i: (i, 0, 0)),
        out_shape=jax.ShapeDtypeStruct((E_PAD // RB, RB, HC), jnp.float32),
    )(rel3, time_w.reshape(1, HC), time_b.reshape(1, HC), edge_w.T)
    return out.reshape(E_PAD, HC)


def _sc_attn_body(qn_hbm, kn_hbm, vn_hbm, em_hbm, src_hbm, dst_hbm, acc_hbm,
                  src_all, dst_all, qA, kA, vA, eA, qB, kB, vB, eB,
                  msg, acc_sh, semA, semB):
    cid = lax.axis_index("c")
    sid = lax.axis_index("s")
    wid = cid * NS + sid
    iota = lax.iota(jnp.int32, L)
    zf = jnp.zeros((L,), jnp.float32)

    pltpu.sync_copy(src_hbm.at[wid], src_all)
    pltpu.sync_copy(dst_hbm.at[wid], dst_all)

    def mrow(i, _):
        for j in range(3):
            msg[i, pl.ds(j * L, L)] = zf
        return 0
    lax.fori_loop(0, B, mrow, 0)
    tbase = sid * ROWS_PER_TILE
    for j in range(ROWS_PER_TILE // B):
        pltpu.sync_copy(msg, acc_sh.at[pl.ds(tbase + j * B, B)])
    plsc.subcore_barrier()

    def issue(blk, bufs, sem):
        qr, kr, vr, er = bufs
        idx = src_all.at[blk]
        pltpu.async_copy(kn_hbm.at[idx], kr, sem)
        pltpu.async_copy(vn_hbm.at[idx], vr, sem)
        pltpu.async_copy(qn_hbm.at[dst_all.at[blk]], qr, sem)
        pltpu.async_copy(em_hbm.at[pl.ds((wid * BLKS + blk) * B, B)], er, sem)

    def drain(blk, bufs, sem):
        qr, kr, vr, er = bufs
        idx = src_all.at[blk]
        pltpu.make_async_copy(kn_hbm.at[idx], kr, sem).wait()
        pltpu.make_async_copy(vn_hbm.at[idx], vr, sem).wait()
        pltpu.make_async_copy(qn_hbm.at[dst_all.at[blk]], qr, sem).wait()
        pltpu.make_async_copy(em_hbm.at[pl.ds((wid * BLKS + blk) * B, B)], er,
                              sem).wait()

    def compute(blk, bufs):
        qr, kr, vr, er = bufs

        def grp_body(g, _):
            ids = g * L + iota
            a0 = zf
            a1 = zf
            for c in range(HC):
                cv = jnp.full((L,), c, jnp.int32)
                qc = plsc.load_gather(qr, [ids, cv])
                kc = plsc.load_gather(kr, [ids, cv])
                ec = plsc.load_gather(er, [ids, cv])
                t = qc * (kc + ec)
                if c < 16:
                    a0 = a0 + t
                else:
                    a1 = a1 + t
            p0 = jnp.exp(a0 * 0.25)
            p1 = jnp.exp(a1 * 0.25)
            for c in range(HC):
                cv = jnp.full((L,), c, jnp.int32)
                vc = plsc.load_gather(vr, [ids, cv])
                ec = plsc.load_gather(er, [ids, cv])
                p = p0 if c < 16 else p1
                plsc.store_scatter(msg, [ids, cv], p * (vc + ec))
            plsc.store_scatter(msg, [ids, jnp.full((L,), 32, jnp.int32)], p0)
            plsc.store_scatter(msg, [ids, jnp.full((L,), 33, jnp.int32)], p1)
            return 0

        lax.fori_loop(0, B // L, grp_body, 0)
        pltpu.sync_copy(msg, acc_sh.at[dst_all.at[blk]], add=True)

    bufsA = (qA, kA, vA, eA)
    bufsB = (qB, kB, vB, eB)
    issue(0, bufsA, semA)

    def pair_body(i, _):
        issue(2 * i + 1, bufsB, semB)
        drain(2 * i, bufsA, semA)
        compute(2 * i, bufsA)
        issue(2 * i + 2, bufsA, semA)
        drain(2 * i + 1, bufsB, semB)
        compute(2 * i + 1, bufsB)
        return 0

    lax.fori_loop(0, (BLKS - 1) // 2, pair_body, 0)
    drain(BLKS - 1, bufsA, semA)
    compute(BLKS - 1, bufsA)

    plsc.subcore_barrier()
    pltpu.sync_copy(acc_sh.at[pl.ds(tbase, ROWS_PER_TILE)],
                    acc_hbm.at[cid, pl.ds(tbase, ROWS_PER_TILE)])


def _sc_attn(qn_pad, kn, vn, e_mat, src_pad, dst_pad):
    mesh = plsc.VectorSubcoreMesh(core_axis_name="c", subcore_axis_name="s")
    rows = lambda: pltpu.VMEM((B, HC), jnp.float32)
    return pl.kernel(
        _sc_attn_body,
        out_type=jax.ShapeDtypeStruct((NC, NP, 48), jnp.float32),
        mesh=mesh,
        scratch_types=[
            pltpu.VMEM((BLKS, B), jnp.int32),
            pltpu.VMEM((BLKS, B), jnp.int32),
            rows(), rows(), rows(), rows(),
            rows(), rows(), rows(), rows(),
            pltpu.VMEM((B, 48), jnp.float32),
            pltpu.VMEM_SHARED((NP, 48), jnp.float32),
            pltpu.SemaphoreType.DMA,
            pltpu.SemaphoreType.DMA,
        ],
        compiler_params=pltpu.CompilerParams(needs_layout_passes=False, use_tc_tiling_on_sc=False),
    )(qn_pad, kn, vn, e_mat,
      src_pad.reshape(NW, BLKS, B), dst_pad.reshape(NW, BLKS, B))


def _tc_final_body(a0_ref, a1_ref, hs_ref, owT, ob, out_ref):
    a = a0_ref[...] + a1_ref[...]
    d0 = a[:, 32:33] + 1e-16
    d1 = a[:, 33:34] + 1e-16
    h0 = a[:, 0:16] / d0
    h1 = a[:, 16:32] / d1
    w = owT[...]
    o = (jnp.dot(h0, w[0:16, :], preferred_element_type=jnp.float32)
         + jnp.dot(h1, w[16:32, :], preferred_element_type=jnp.float32)
         + jnp.dot(hs_ref[...], w, preferred_element_type=jnp.float32)
         + ob[...])
    m = jnp.max(o, axis=1, keepdims=True)
    s = o - m
    out_ref[...] = s - jnp.log(jnp.sum(jnp.exp(s), axis=1, keepdims=True))


def _tc_final(acc, hs, out_w, out_b):
    R = 2000
    grid = (N // R,)
    return pl.pallas_call(
        _tc_final_body,
        grid=grid,
        in_specs=[pl.BlockSpec((R, 48), lambda i: (i, 0)),
                  pl.BlockSpec((R, 48), lambda i: (i, 0)),
                  pl.BlockSpec((R, HC), lambda i: (i, 0)),
                  pl.BlockSpec((HC, NCLS), lambda i: (0, 0)),
                  pl.BlockSpec((1, NCLS), lambda i: (0, 0))],
        out_specs=pl.BlockSpec((R, NCLS), lambda i: (i, 0)),
        out_shape=jax.ShapeDtypeStruct((N, NCLS), jnp.float32),
    )(acc[0, :N], acc[1, :N], hs, out_w.T, out_b.reshape(1, NCLS))


def _sc_test(table, idx):
    V, D = table.shape
    BT = idx.shape[0]
    b_per_w = BT // NW
    mesh = plsc.VectorSubcoreMesh(core_axis_name="c", subcore_axis_name="s")

    @functools.partial(
        pl.kernel, mesh=mesh,
        out_type=jax.ShapeDtypeStruct((BT, D), jnp.float32),
        scratch_types=[
            pltpu.VMEM((b_per_w,), jnp.int32),
            pltpu.VMEM((b_per_w, D), jnp.float32),
            pltpu.SemaphoreType.DMA,
        ],
    )
    def k(table_hbm, idx_hbm, out_hbm, idx_v, rows_v, sem):
        wid = lax.axis_index("s") * NC + lax.axis_index("c")
        base = wid * b_per_w
        pltpu.sync_copy(idx_hbm.at[pl.ds(base, b_per_w)], idx_v)
        pltpu.async_copy(table_hbm.at[idx_v], rows_v, sem).wait()
        pltpu.sync_copy(rows_v, out_hbm.at[pl.ds(base, b_per_w)])

    return k(table, idx)


def _jnp_control(x, edge_index, node_time, edge_time, time_w, time_b, lin_w,
                 lin_b, key_w, key_b, query_w, query_b, value_w, value_b,
                 edge_w, skip_w, skip_b, out_w, out_b):
    src = edge_index[0]
    dst = edge_index[1]
    n = x.shape[0]
    rel_t = node_time[src][:, None] - edge_time
    rel_t_enc = jnp.cos(rel_t @ time_w.T + time_b)
    h1 = jax.nn.relu(x @ lin_w.T + lin_b)
    q = (h1 @ query_w.T + query_b).reshape(n, 2, 16)[dst]
    k = (h1 @ key_w.T + key_b).reshape(n, 2, 16)[src]
    v = (h1 @ value_w.T + value_b).reshape(n, 2, 16)[src]
    e = (rel_t_enc @ edge_w.T).reshape(-1, 2, 16)
    k = k + e
    v = v + e
    alpha = (q * k).sum(-1) * 0.25
    amax = jax.ops.segment_max(alpha, dst, num_segments=n)
    amax = jnp.where(jnp.isfinite(amax), amax, 0.0)
    alpha = jnp.exp(alpha - amax[dst])
    denom = jax.ops.segment_sum(alpha, dst, num_segments=n)
    alpha = alpha / (denom[dst] + 1e-16)
    msg = v * alpha[:, :, None]
    agg = jax.ops.segment_sum(msg, dst, num_segments=n)
    h = agg.reshape(n, 32) + (h1 @ skip_w.T + skip_b)
    out = h @ out_w.T + out_b
    return jax.nn.log_softmax(out, axis=1)


def _pipeline(x, edge_index, node_time, edge_time, time_w, time_b, lin_w,
              lin_b, key_w, key_b, query_w, query_b, value_w, value_b,
              edge_w, skip_w, skip_b, out_w, out_b):
    src = edge_index[0]
    dst = edge_index[1]
    pad = E_PAD - E
    src_pad = jnp.concatenate([src, jnp.zeros((pad,), jnp.int32)])
    dst_pad = jnp.concatenate([dst, jnp.full((pad,), N, jnp.int32)])
    et_pad = jnp.concatenate([edge_time.reshape(E), jnp.zeros((pad,), jnp.float32)])

    qn, kn, vn, hs = _tc_proj(x, lin_w, lin_b, key_w, key_b, query_w, query_b,
                              value_w, value_b, skip_w, skip_b)
    qn_pad = jnp.concatenate([qn, jnp.zeros((QROWS - N, HC), jnp.float32)])

    rel_pad = _sc_relt(node_time, src_pad, et_pad)
    e_mat = _tc_emat(rel_pad, time_w, time_b, edge_w)
    acc = _sc_attn(qn_pad, kn, vn, e_mat, src_pad, dst_pad)
    return _tc_final(acc, hs, out_w, out_b)


def kernel(x, edge_index, node_time, edge_time, time_w, time_b, lin_w, lin_b,
           key_w, key_b, query_w, query_b, value_w, value_b, edge_w,
           skip_w, skip_b, out_w, out_b):
    return _pipeline(x, edge_index, node_time, edge_time, time_w, time_b,
                     lin_w, lin_b, key_w, key_b, query_w, query_b, value_w,
                     value_b, edge_w, skip_w, skip_b, out_w, out_b)


# speedup vs baseline: 14.8995x; 1.0361x over previous
import functools

import jax
import jax.numpy as jnp
from jax import lax
from jax.experimental import pallas as pl
from jax.experimental.pallas import tpu as pltpu
from jax.experimental.pallas import tpu_sc as plsc

N = 10000
E = 320000
D_IN = 128
HC = 32
NCLS = 2

NC = 2
NS = 16
NW = NC * NS
L = 16

B = 128
BLKS = -(-E // (NW * B))
EPW = BLKS * B
E_PAD = NW * EPW

ROWS_PER_TILE = 640
NP = NS * ROWS_PER_TILE
QROWS = N + L


def _tc_proj_body(x_ref, lwT, lb, qwT, qb, kwT, kb, vwT, vb, swT, sb,
                  qn, kn, vn, hs):
    xb = x_ref[...]
    h1 = jnp.maximum(jnp.dot(xb, lwT[...], preferred_element_type=jnp.float32)
                     + lb[...], 0.0)
    qn[...] = jnp.dot(h1, qwT[...], preferred_element_type=jnp.float32) + qb[...]
    kn[...] = jnp.dot(h1, kwT[...], preferred_element_type=jnp.float32) + kb[...]
    vn[...] = jnp.dot(h1, vwT[...], preferred_element_type=jnp.float32) + vb[...]
    hs[...] = jnp.dot(h1, swT[...], preferred_element_type=jnp.float32) + sb[...]


def _tc_proj(x, lin_w, lin_b, key_w, key_b, query_w, query_b, value_w, value_b,
             skip_w, skip_b):
    R = 2000
    grid = (N // R,)
    row_spec = pl.BlockSpec((R, D_IN), lambda i: (i, 0))
    out_spec = pl.BlockSpec((R, HC), lambda i: (i, 0))
    full = lambda shape: pl.BlockSpec(shape, lambda i: (0, 0))
    return pl.pallas_call(
        _tc_proj_body,
        grid=grid,
        in_specs=[row_spec,
                  full((D_IN, HC)), full((1, HC)),
                  full((HC, HC)), full((1, HC)),
                  full((HC, HC)), full((1, HC)),
                  full((HC, HC)), full((1, HC)),
                  full((HC, HC)), full((1, HC))],
        out_specs=[out_spec] * 4,
        out_shape=[jax.ShapeDtypeStruct((N, HC), jnp.float32)] * 4,
    )(x, lin_w.T, lin_b.reshape(1, HC),
      query_w.T, query_b.reshape(1, HC),
      key_w.T, key_b.reshape(1, HC),
      value_w.T, value_b.reshape(1, HC),
      skip_w.T, skip_b.reshape(1, HC))


def _sc_relt_body(nt_hbm, src_hbm, et_hbm, rel_hbm, nt_v, src_v, et_v, rel_v):
    cid = lax.axis_index("c")
    sid = lax.axis_index("s")
    wid = cid * NS + sid
    wbase = wid * EPW
    pltpu.sync_copy(nt_hbm, nt_v)
    pltpu.sync_copy(src_hbm.at[pl.ds(wbase, EPW)], src_v)
    pltpu.sync_copy(et_hbm.at[pl.ds(wbase, EPW)], et_v)

    def grp_body(g, _):
        off = g * L
        sidx = src_v[pl.ds(off, L)]
        r = plsc.load_gather(nt_v, [sidx]) - et_v[pl.ds(off, L)]
        rel_v[pl.ds(off, L)] = r
        return 0

    lax.fori_loop(0, EPW // L, grp_body, 0)
    pltpu.sync_copy(rel_v, rel_hbm.at[pl.ds(wbase, EPW)])


def _sc_relt(node_time, src_pad, et_pad):
    mesh = plsc.VectorSubcoreMesh(core_axis_name="c", subcore_axis_name="s")
    return pl.kernel(
        _sc_relt_body,
        out_type=jax.ShapeDtypeStruct((E_PAD,), jnp.float32),
        mesh=mesh,
        scratch_types=[
            pltpu.VMEM((N,), jnp.float32),
            pltpu.VMEM((EPW,), jnp.int32),
            pltpu.VMEM((EPW,), jnp.float32),
            pltpu.VMEM((EPW,), jnp.float32),
        ],
        compiler_params=pltpu.CompilerParams(needs_layout_passes=False, use_tc_tiling_on_sc=False),
    )(node_time, src_pad, et_pad)


def _tc_emat_body(rel_ref, twr, tbr, ewT, out_ref):
    r = rel_ref[0]
    enc = jnp.cos(r * twr[...] + tbr[...])
    out_ref[0] = jnp.dot(enc, ewT[...], preferred_element_type=jnp.float32)


def _tc_emat(rel_pad, time_w, time_b, edge_w):
    RB = 1024
    grid = (E_PAD // RB,)
    rel3 = rel_pad.reshape(E_PAD // RB, RB, 1)
    out = pl.pallas_call(
        _tc_emat_body,
        grid=grid,
        in_specs=[pl.BlockSpec((1, RB, 1), lambda i: (i, 0, 0)),
                  pl.BlockSpec((1, HC), lambda i: (0, 0)),
                  pl.BlockSpec((1, HC), lambda i: (0, 0)),
                  pl.BlockSpec((HC, HC), lambda i: (0, 0))],
        out_specs=pl.BlockSpec((1, RB, HC), lambda i: (i, 0, 0)),
        out_shape=jax.ShapeDtypeStruct((E_PAD // RB, RB, HC), jnp.float32),
    )(rel3, time_w.reshape(1, HC), time_b.reshape(1, HC), edge_w.T)
    return out.reshape(E_PAD, HC)


def _sc_attn_body(qn_hbm, kn_hbm, vn_hbm, em_hbm, src_hbm, dst_hbm, acc_hbm,
                  src_all, dst_all, qA, kA, vA, eA, qB, kB, vB, eB,
                  msg, acc_sh, semA, semB):
    cid = lax.axis_index("c")
    sid = lax.axis_index("s")
    wid = cid * NS + sid
    iota = lax.iota(jnp.int32, L)
    zf = jnp.zeros((L,), jnp.float32)

    pltpu.sync_copy(src_hbm.at[wid], src_all)
    pltpu.sync_copy(dst_hbm.at[wid], dst_all)

    def mrow(i, _):
        for j in range(3):
            msg[i, pl.ds(j * L, L)] = zf
        return 0
    lax.fori_loop(0, B, mrow, 0)
    tbase = sid * ROWS_PER_TILE
    for j in range(ROWS_PER_TILE // B):
        pltpu.sync_copy(msg, acc_sh.at[pl.ds(tbase + j * B, B)])
    plsc.subcore_barrier()

    def issue(blk, bufs, sem):
        qr, kr, vr, er = bufs
        idx = src_all.at[blk]
        pltpu.async_copy(kn_hbm.at[idx], kr, sem)
        pltpu.async_copy(vn_hbm.at[idx], vr, sem)
        pltpu.async_copy(qn_hbm.at[dst_all.at[blk]], qr, sem)
        pltpu.async_copy(em_hbm.at[pl.ds((wid * BLKS + blk) * B, B)], er, sem)

    def drain(blk, bufs, sem):
        qr, kr, vr, er = bufs
        idx = src_all.at[blk]
        pltpu.make_async_copy(kn_hbm.at[idx], kr, sem).wait()
        pltpu.make_async_copy(vn_hbm.at[idx], vr, sem).wait()
        pltpu.make_async_copy(qn_hbm.at[dst_all.at[blk]], qr, sem).wait()
        pltpu.make_async_copy(em_hbm.at[pl.ds((wid * BLKS + blk) * B, B)], er,
                              sem).wait()

    def compute(blk, bufs):
        qr, kr, vr, er = bufs

        def grp_body(g, _):
            ids = g * L + iota
            a0 = zf
            a1 = zf
            for c in range(HC):
                cv = jnp.full((L,), c, jnp.int32)
                qc = plsc.load_gather(qr, [ids, cv])
                kc = plsc.load_gather(kr, [ids, cv])
                ec = plsc.load_gather(er, [ids, cv])
                t = qc * (kc + ec)
                if c < 16:
                    a0 = a0 + t
                else:
                    a1 = a1 + t
            p0 = jnp.exp(a0 * 0.25)
            p1 = jnp.exp(a1 * 0.25)
            for c in range(HC):
                cv = jnp.full((L,), c, jnp.int32)
                vc = plsc.load_gather(vr, [ids, cv])
                ec = plsc.load_gather(er, [ids, cv])
                p = p0 if c < 16 else p1
                plsc.store_scatter(msg, [ids, cv], p * (vc + ec))
            plsc.store_scatter(msg, [ids, jnp.full((L,), 32, jnp.int32)], p0)
            plsc.store_scatter(msg, [ids, jnp.full((L,), 33, jnp.int32)], p1)
            return 0

        lax.fori_loop(0, B // L, grp_body, 0)
        pltpu.sync_copy(msg, acc_sh.at[dst_all.at[blk]], add=True)

    bufsA = (qA, kA, vA, eA)
    bufsB = (qB, kB, vB, eB)
    issue(0, bufsA, semA)

    def pair_body(i, _):
        issue(2 * i + 1, bufsB, semB)
        drain(2 * i, bufsA, semA)
        compute(2 * i, bufsA)
        issue(2 * i + 2, bufsA, semA)
        drain(2 * i + 1, bufsB, semB)
        compute(2 * i + 1, bufsB)
        return 0

    lax.fori_loop(0, (BLKS - 1) // 2, pair_body, 0)
    drain(BLKS - 1, bufsA, semA)
    compute(BLKS - 1, bufsA)

    plsc.subcore_barrier()
    pltpu.sync_copy(acc_sh.at[pl.ds(tbase, ROWS_PER_TILE)],
                    acc_hbm.at[cid, pl.ds(tbase, ROWS_PER_TILE)])


def _sc_attn(qn_pad, kn, vn, e_mat, src_pad, dst_pad):
    mesh = plsc.VectorSubcoreMesh(core_axis_name="c", subcore_axis_name="s")
    rows = lambda: pltpu.VMEM((B, HC), jnp.float32)
    return pl.kernel(
        _sc_attn_body,
        out_type=jax.ShapeDtypeStruct((NC, NP, 48), jnp.float32),
        mesh=mesh,
        scratch_types=[
            pltpu.VMEM((BLKS, B), jnp.int32),
            pltpu.VMEM((BLKS, B), jnp.int32),
            rows(), rows(), rows(), rows(),
            rows(), rows(), rows(), rows(),
            pltpu.VMEM((B, 48), jnp.float32),
            pltpu.VMEM_SHARED((NP, 48), jnp.float32),
            pltpu.SemaphoreType.DMA,
            pltpu.SemaphoreType.DMA,
        ],
        compiler_params=pltpu.CompilerParams(needs_layout_passes=False, use_tc_tiling_on_sc=False),
    )(qn_pad, kn, vn, e_mat,
      src_pad.reshape(NW, BLKS, B), dst_pad.reshape(NW, BLKS, B))


def _tc_final_body(a0_ref, a1_ref, hs_ref, owT, ob, out_ref):
    a = a0_ref[...] + a1_ref[...]
    d0 = a[:, 32:33] + 1e-16
    d1 = a[:, 33:34] + 1e-16
    h0 = a[:, 0:16] / d0
    h1 = a[:, 16:32] / d1
    w = owT[...]
    o = (jnp.dot(h0, w[0:16, :], preferred_element_type=jnp.float32)
         + jnp.dot(h1, w[16:32, :], preferred_element_type=jnp.float32)
         + jnp.dot(hs_ref[...], w, preferred_element_type=jnp.float32)
         + ob[...])
    m = jnp.max(o, axis=1, keepdims=True)
    s = o - m
    out_ref[...] = s - jnp.log(jnp.sum(jnp.exp(s), axis=1, keepdims=True))


def _tc_final(acc, hs, out_w, out_b):
    R = 2000
    grid = (N // R,)
    return pl.pallas_call(
        _tc_final_body,
        grid=grid,
        in_specs=[pl.BlockSpec((R, 48), lambda i: (i, 0)),
                  pl.BlockSpec((R, 48), lambda i: (i, 0)),
                  pl.BlockSpec((R, HC), lambda i: (i, 0)),
                  pl.BlockSpec((HC, NCLS), lambda i: (0, 0)),
                  pl.BlockSpec((1, NCLS), lambda i: (0, 0))],
        out_specs=pl.BlockSpec((R, NCLS), lambda i: (i, 0)),
        out_shape=jax.ShapeDtypeStruct((N, NCLS), jnp.float32),
    )(acc[0, :N], acc[1, :N], hs, out_w.T, out_b.reshape(1, NCLS))


def _sc_test(table, idx):
    V, D = table.shape
    BT = idx.shape[0]
    b_per_w = BT // NW
    mesh = plsc.VectorSubcoreMesh(core_axis_name="c", subcore_axis_name="s")

    @functools.partial(
        pl.kernel, mesh=mesh,
        out_type=jax.ShapeDtypeStruct((BT, D), jnp.float32),
        scratch_types=[
            pltpu.VMEM((b_per_w,), jnp.int32),
            pltpu.VMEM((b_per_w, D), jnp.float32),
            pltpu.SemaphoreType.DMA,
        ],
    )
    def k(table_hbm, idx_hbm, out_hbm, idx_v, rows_v, sem):
        wid = lax.axis_index("s") * NC + lax.axis_index("c")
        base = wid * b_per_w
        pltpu.sync_copy(idx_hbm.at[pl.ds(base, b_per_w)], idx_v)
        pltpu.async_copy(table_hbm.at[idx_v], rows_v, sem).wait()
        pltpu.sync_copy(rows_v, out_hbm.at[pl.ds(base, b_per_w)])

    return k(table, idx)


def _jnp_control(x, edge_index, node_time, edge_time, time_w, time_b, lin_w,
                 lin_b, key_w, key_b, query_w, query_b, value_w, value_b,
                 edge_w, skip_w, skip_b, out_w, out_b):
    src = edge_index[0]
    dst = edge_index[1]
    n = x.shape[0]
    rel_t = node_time[src][:, None] - edge_time
    rel_t_enc = jnp.cos(rel_t @ time_w.T + time_b)
    h1 = jax.nn.relu(x @ lin_w.T + lin_b)
    q = (h1 @ query_w.T + query_b).reshape(n, 2, 16)[dst]
    k = (h1 @ key_w.T + key_b).reshape(n, 2, 16)[src]
    v = (h1 @ value_w.T + value_b).reshape(n, 2, 16)[src]
    e = (rel_t_enc @ edge_w.T).reshape(-1, 2, 16)
    k = k + e
    v = v + e
    alpha = (q * k).sum(-1) * 0.25
    amax = jax.ops.segment_max(alpha, dst, num_segments=n)
    amax = jnp.where(jnp.isfinite(amax), amax, 0.0)
    alpha = jnp.exp(alpha - amax[dst])
    denom = jax.ops.segment_sum(alpha, dst, num_segments=n)
    alpha = alpha / (denom[dst] + 1e-16)
    msg = v * alpha[:, :, None]
    agg = jax.ops.segment_sum(msg, dst, num_segments=n)
    h = agg.reshape(n, 32) + (h1 @ skip_w.T + skip_b)
    out = h @ out_w.T + out_b
    return jax.nn.log_softmax(out, axis=1)


def _pipeline(x, edge_index, node_time, edge_time, time_w, time_b, lin_w,
              lin_b, key_w, key_b, query_w, query_b, value_w, value_b,
              edge_w, skip_w, skip_b, out_w, out_b):
    src = edge_index[0]
    dst = edge_index[1]
    pad = E_PAD - E
    src_pad = jnp.concatenate([src, jnp.zeros((pad,), jnp.int32)])
    dst_pad = jnp.concatenate([dst, jnp.full((pad,), N, jnp.int32)])
    et_pad = jnp.concatenate([edge_time.reshape(E), jnp.zeros((pad,), jnp.float32)])

    qn, kn, vn, hs = _tc_proj(x, lin_w, lin_b, key_w, key_b, query_w, query_b,
                              value_w, value_b, skip_w, skip_b)
    qn_pad = jnp.concatenate([qn, jnp.zeros((QROWS - N, HC), jnp.float32)])

    rel_pad = _sc_relt(node_time, src_pad, et_pad)
    e_mat = _tc_emat(rel_pad, time_w, time_b, edge_w)
    acc = _sc_attn(qn_pad, kn, vn, e_mat, src_pad, dst_pad)
    return _tc_final(acc, hs, out_w, out_b)


def kernel(x, edge_index, node_time, edge_time, time_w, time_b, lin_w, lin_b,
           key_w, key_b, query_w, query_b, value_w, value_b, edge_w,
           skip_w, skip_b, out_w, out_b):
    return _pipeline(x, edge_index, node_time, edge_time, time_w, time_b,
                     lin_w, lin_b, key_w, key_b, query_w, query_b, value_w,
                     value_b, edge_w, skip_w, skip_b, out_w, out_b)


# speedup vs baseline: 14.9084x; 1.0006x over previous
import functools

import jax
import jax.numpy as jnp
from jax import lax
from jax.experimental import pallas as pl
from jax.experimental.pallas import tpu as pltpu
from jax.experimental.pallas import tpu_sc as plsc

N = 10000
E = 320000
D_IN = 128
HC = 32
NCLS = 2

NC = 2
NS = 16
NW = NC * NS
L = 16

B = 128
BLKS = -(-E // (NW * B))
EPW = BLKS * B
E_PAD = NW * EPW

ROWS_PER_TILE = 640
NP = NS * ROWS_PER_TILE
QROWS = N + L


def _tc_proj_body(x_ref, lwT, lb, qwT, qb, kwT, kb, vwT, vb, swT, sb,
                  qn, kn, vn, hs):
    xb = x_ref[...]
    h1 = jnp.maximum(jnp.dot(xb, lwT[...], preferred_element_type=jnp.float32)
                     + lb[...], 0.0)
    qn[...] = jnp.dot(h1, qwT[...], preferred_element_type=jnp.float32) + qb[...]
    kn[...] = jnp.dot(h1, kwT[...], preferred_element_type=jnp.float32) + kb[...]
    vn[...] = jnp.dot(h1, vwT[...], preferred_element_type=jnp.float32) + vb[...]
    hs[...] = jnp.dot(h1, swT[...], preferred_element_type=jnp.float32) + sb[...]


def _tc_proj(x, lin_w, lin_b, key_w, key_b, query_w, query_b, value_w, value_b,
             skip_w, skip_b):
    R = 2000
    grid = (N // R,)
    row_spec = pl.BlockSpec((R, D_IN), lambda i: (i, 0))
    out_spec = pl.BlockSpec((R, HC), lambda i: (i, 0))
    full = lambda shape: pl.BlockSpec(shape, lambda i: (0, 0))
    return pl.pallas_call(
        _tc_proj_body,
        grid=grid,
        in_specs=[row_spec,
                  full((D_IN, HC)), full((1, HC)),
                  full((HC, HC)), full((1, HC)),
                  full((HC, HC)), full((1, HC)),
                  full((HC, HC)), full((1, HC)),
                  full((HC, HC)), full((1, HC))],
        out_specs=[out_spec] * 4,
        out_shape=[jax.ShapeDtypeStruct((N, HC), jnp.float32)] * 4,
    )(x, lin_w.T, lin_b.reshape(1, HC),
      query_w.T, query_b.reshape(1, HC),
      key_w.T, key_b.reshape(1, HC),
      value_w.T, value_b.reshape(1, HC),
      skip_w.T, skip_b.reshape(1, HC))


def _sc_relt_body(nt_hbm, src_hbm, et_hbm, rel_hbm, nt_v, src_v, et_v, rel_v):
    cid = lax.axis_index("c")
    sid = lax.axis_index("s")
    wid = cid * NS + sid
    wbase = wid * EPW
    pltpu.sync_copy(nt_hbm, nt_v)
    pltpu.sync_copy(src_hbm.at[pl.ds(wbase, EPW)], src_v)
    pltpu.sync_copy(et_hbm.at[pl.ds(wbase, EPW)], et_v)

    def grp_body(g, _):
        off = g * L
        sidx = src_v[pl.ds(off, L)]
        r = plsc.load_gather(nt_v, [sidx]) - et_v[pl.ds(off, L)]
        rel_v[pl.ds(off, L)] = r
        return 0

    lax.fori_loop(0, EPW // L, grp_body, 0)
    pltpu.sync_copy(rel_v, rel_hbm.at[pl.ds(wbase, EPW)])


def _sc_relt(node_time, src_pad, et_pad):
    mesh = plsc.VectorSubcoreMesh(core_axis_name="c", subcore_axis_name="s")
    return pl.kernel(
        _sc_relt_body,
        out_type=jax.ShapeDtypeStruct((E_PAD,), jnp.float32),
        mesh=mesh,
        scratch_types=[
            pltpu.VMEM((N,), jnp.float32),
            pltpu.VMEM((EPW,), jnp.int32),
            pltpu.VMEM((EPW,), jnp.float32),
            pltpu.VMEM((EPW,), jnp.float32),
        ],
        compiler_params=pltpu.CompilerParams(needs_layout_passes=False, use_tc_tiling_on_sc=False),
    )(node_time, src_pad, et_pad)


def _tc_emat_body(rel_ref, twr, tbr, ewT, out_ref):
    r = rel_ref[0]
    enc = jnp.cos(r * twr[...] + tbr[...])
    out_ref[0] = jnp.dot(enc, ewT[...], preferred_element_type=jnp.float32)


def _tc_emat(rel_pad, time_w, time_b, edge_w):
    RB = 1024
    grid = (E_PAD // RB,)
    rel3 = rel_pad.reshape(E_PAD // RB, RB, 1)
    out = pl.pallas_call(
        _tc_emat_body,
        grid=grid,
        in_specs=[pl.BlockSpec((1, RB, 1), lambda i: (i, 0, 0)),
                  pl.BlockSpec((1, HC), lambda i: (0, 0)),
                  pl.BlockSpec((1, HC), lambda i: (0, 0)),
                  pl.BlockSpec((HC, HC), lambda i: (0, 0))],
        out_specs=pl.BlockSpec((1, RB, HC), lambda i: (i, 0, 0)),
        out_shape=jax.ShapeDtypeStruct((E_PAD // RB, RB, HC), jnp.float32),
    )(rel3, time_w.reshape(1, HC), time_b.reshape(1, HC), edge_w.T)
    return out.reshape(E_PAD, HC)


def _sc_attn_body(qn_hbm, kn_hbm, vn_hbm, em_hbm, src_hbm, dst_hbm, acc_hbm,
                  src_all, dst_all, qA, kA, vA, eA, qB, kB, vB, eB,
                  msg, acc_sh, semA, semB):
    cid = lax.axis_index("c")
    sid = lax.axis_index("s")
    wid = cid * NS + sid
    iota = lax.iota(jnp.int32, L)
    zf = jnp.zeros((L,), jnp.float32)

    pltpu.sync_copy(src_hbm.at[wid], src_all)
    pltpu.sync_copy(dst_hbm.at[wid], dst_all)

    def mrow(i, _):
        for j in range(3):
            msg[i, pl.ds(j * L, L)] = zf
        return 0
    lax.fori_loop(0, B, mrow, 0)
    tbase = sid * ROWS_PER_TILE
    for j in range(ROWS_PER_TILE // B):
        pltpu.sync_copy(msg, acc_sh.at[pl.ds(tbase + j * B, B)])
    plsc.subcore_barrier()

    def issue(blk, bufs, sem):
        qr, kr, vr, er = bufs
        idx = src_all.at[blk]
        pltpu.async_copy(kn_hbm.at[idx], kr, sem)
        pltpu.async_copy(vn_hbm.at[idx], vr, sem)
        pltpu.async_copy(qn_hbm.at[dst_all.at[blk]], qr, sem)
        pltpu.async_copy(em_hbm.at[pl.ds((wid * BLKS + blk) * B, B)], er, sem)

    def drain(blk, bufs, sem):
        qr, kr, vr, er = bufs
        idx = src_all.at[blk]
        pltpu.make_async_copy(kn_hbm.at[idx], kr, sem).wait()
        pltpu.make_async_copy(vn_hbm.at[idx], vr, sem).wait()
        pltpu.make_async_copy(qn_hbm.at[dst_all.at[blk]], qr, sem).wait()
        pltpu.make_async_copy(em_hbm.at[pl.ds((wid * BLKS + blk) * B, B)], er,
                              sem).wait()

    def compute(blk, bufs):
        qr, kr, vr, er = bufs

        def grp_body(g, _):
            ids = g * L + iota
            a0 = zf
            a1 = zf
            for c in range(HC):
                cv = jnp.full((L,), c, jnp.int32)
                qc = plsc.load_gather(qr, [ids, cv])
                kc = plsc.load_gather(kr, [ids, cv])
                ec = plsc.load_gather(er, [ids, cv])
                t = qc * (kc + ec)
                if c < 16:
                    a0 = a0 + t
                else:
                    a1 = a1 + t
            p0 = jnp.exp(a0 * 0.25)
            p1 = jnp.exp(a1 * 0.25)
            for c in range(HC):
                cv = jnp.full((L,), c, jnp.int32)
                vc = plsc.load_gather(vr, [ids, cv])
                ec = plsc.load_gather(er, [ids, cv])
                p = p0 if c < 16 else p1
                plsc.store_scatter(msg, [ids, cv], p * (vc + ec))
            plsc.store_scatter(msg, [ids, jnp.full((L,), 32, jnp.int32)], p0)
            plsc.store_scatter(msg, [ids, jnp.full((L,), 33, jnp.int32)], p1)
            return 0

        def grp2(h, _):
            grp_body(2 * h, 0)
            grp_body(2 * h + 1, 0)
            return 0

        lax.fori_loop(0, B // L // 2, grp2, 0)
        pltpu.sync_copy(msg, acc_sh.at[dst_all.at[blk]], add=True)

    bufsA = (qA, kA, vA, eA)
    bufsB = (qB, kB, vB, eB)
    issue(0, bufsA, semA)

    def pair_body(i, _):
        issue(2 * i + 1, bufsB, semB)
        drain(2 * i, bufsA, semA)
        compute(2 * i, bufsA)
        issue(2 * i + 2, bufsA, semA)
        drain(2 * i + 1, bufsB, semB)
        compute(2 * i + 1, bufsB)
        return 0

    lax.fori_loop(0, (BLKS - 1) // 2, pair_body, 0)
    drain(BLKS - 1, bufsA, semA)
    compute(BLKS - 1, bufsA)

    plsc.subcore_barrier()
    pltpu.sync_copy(acc_sh.at[pl.ds(tbase, ROWS_PER_TILE)],
                    acc_hbm.at[cid, pl.ds(tbase, ROWS_PER_TILE)])


def _sc_attn(qn_pad, kn, vn, e_mat, src_pad, dst_pad):
    mesh = plsc.VectorSubcoreMesh(core_axis_name="c", subcore_axis_name="s")
    rows = lambda: pltpu.VMEM((B, HC), jnp.float32)
    return pl.kernel(
        _sc_attn_body,
        out_type=jax.ShapeDtypeStruct((NC, NP, 48), jnp.float32),
        mesh=mesh,
        scratch_types=[
            pltpu.VMEM((BLKS, B), jnp.int32),
            pltpu.VMEM((BLKS, B), jnp.int32),
            rows(), rows(), rows(), rows(),
            rows(), rows(), rows(), rows(),
            pltpu.VMEM((B, 48), jnp.float32),
            pltpu.VMEM_SHARED((NP, 48), jnp.float32),
            pltpu.SemaphoreType.DMA,
            pltpu.SemaphoreType.DMA,
        ],
        compiler_params=pltpu.CompilerParams(needs_layout_passes=False, use_tc_tiling_on_sc=False),
    )(qn_pad, kn, vn, e_mat,
      src_pad.reshape(NW, BLKS, B), dst_pad.reshape(NW, BLKS, B))


def _tc_final_body(a0_ref, a1_ref, hs_ref, owT, ob, out_ref):
    a = a0_ref[...] + a1_ref[...]
    d0 = a[:, 32:33] + 1e-16
    d1 = a[:, 33:34] + 1e-16
    h0 = a[:, 0:16] / d0
    h1 = a[:, 16:32] / d1
    w = owT[...]
    o = (jnp.dot(h0, w[0:16, :], preferred_element_type=jnp.float32)
         + jnp.dot(h1, w[16:32, :], preferred_element_type=jnp.float32)
         + jnp.dot(hs_ref[...], w, preferred_element_type=jnp.float32)
         + ob[...])
    m = jnp.max(o, axis=1, keepdims=True)
    s = o - m
    out_ref[...] = s - jnp.log(jnp.sum(jnp.exp(s), axis=1, keepdims=True))


def _tc_final(acc, hs, out_w, out_b):
    R = 2000
    grid = (N // R,)
    return pl.pallas_call(
        _tc_final_body,
        grid=grid,
        in_specs=[pl.BlockSpec((R, 48), lambda i: (i, 0)),
                  pl.BlockSpec((R, 48), lambda i: (i, 0)),
                  pl.BlockSpec((R, HC), lambda i: (i, 0)),
                  pl.BlockSpec((HC, NCLS), lambda i: (0, 0)),
                  pl.BlockSpec((1, NCLS), lambda i: (0, 0))],
        out_specs=pl.BlockSpec((R, NCLS), lambda i: (i, 0)),
        out_shape=jax.ShapeDtypeStruct((N, NCLS), jnp.float32),
    )(acc[0, :N], acc[1, :N], hs, out_w.T, out_b.reshape(1, NCLS))


def _sc_test(table, idx):
    V, D = table.shape
    BT = idx.shape[0]
    b_per_w = BT // NW
    mesh = plsc.VectorSubcoreMesh(core_axis_name="c", subcore_axis_name="s")

    @functools.partial(
        pl.kernel, mesh=mesh,
        out_type=jax.ShapeDtypeStruct((BT, D), jnp.float32),
        scratch_types=[
            pltpu.VMEM((b_per_w,), jnp.int32),
            pltpu.VMEM((b_per_w, D), jnp.float32),
            pltpu.SemaphoreType.DMA,
        ],
    )
    def k(table_hbm, idx_hbm, out_hbm, idx_v, rows_v, sem):
        wid = lax.axis_index("s") * NC + lax.axis_index("c")
        base = wid * b_per_w
        pltpu.sync_copy(idx_hbm.at[pl.ds(base, b_per_w)], idx_v)
        pltpu.async_copy(table_hbm.at[idx_v], rows_v, sem).wait()
        pltpu.sync_copy(rows_v, out_hbm.at[pl.ds(base, b_per_w)])

    return k(table, idx)


def _jnp_control(x, edge_index, node_time, edge_time, time_w, time_b, lin_w,
                 lin_b, key_w, key_b, query_w, query_b, value_w, value_b,
                 edge_w, skip_w, skip_b, out_w, out_b):
    src = edge_index[0]
    dst = edge_index[1]
    n = x.shape[0]
    rel_t = node_time[src][:, None] - edge_time
    rel_t_enc = jnp.cos(rel_t @ time_w.T + time_b)
    h1 = jax.nn.relu(x @ lin_w.T + lin_b)
    q = (h1 @ query_w.T + query_b).reshape(n, 2, 16)[dst]
    k = (h1 @ key_w.T + key_b).reshape(n, 2, 16)[src]
    v = (h1 @ value_w.T + value_b).reshape(n, 2, 16)[src]
    e = (rel_t_enc @ edge_w.T).reshape(-1, 2, 16)
    k = k + e
    v = v + e
    alpha = (q * k).sum(-1) * 0.25
    amax = jax.ops.segment_max(alpha, dst, num_segments=n)
    amax = jnp.where(jnp.isfinite(amax), amax, 0.0)
    alpha = jnp.exp(alpha - amax[dst])
    denom = jax.ops.segment_sum(alpha, dst, num_segments=n)
    alpha = alpha / (denom[dst] + 1e-16)
    msg = v * alpha[:, :, None]
    agg = jax.ops.segment_sum(msg, dst, num_segments=n)
    h = agg.reshape(n, 32) + (h1 @ skip_w.T + skip_b)
    out = h @ out_w.T + out_b
    return jax.nn.log_softmax(out, axis=1)


def _pipeline(x, edge_index, node_time, edge_time, time_w, time_b, lin_w,
              lin_b, key_w, key_b, query_w, query_b, value_w, value_b,
              edge_w, skip_w, skip_b, out_w, out_b):
    src = edge_index[0]
    dst = edge_index[1]
    pad = E_PAD - E
    src_pad = jnp.concatenate([src, jnp.zeros((pad,), jnp.int32)])
    dst_pad = jnp.concatenate([dst, jnp.full((pad,), N, jnp.int32)])
    et_pad = jnp.concatenate([edge_time.reshape(E), jnp.zeros((pad,), jnp.float32)])

    qn, kn, vn, hs = _tc_proj(x, lin_w, lin_b, key_w, key_b, query_w, query_b,
                              value_w, value_b, skip_w, skip_b)
    qn_pad = jnp.concatenate([qn, jnp.zeros((QROWS - N, HC), jnp.float32)])

    rel_pad = _sc_relt(node_time, src_pad, et_pad)
    e_mat = _tc_emat(rel_pad, time_w, time_b, edge_w)
    acc = _sc_attn(qn_pad, kn, vn, e_mat, src_pad, dst_pad)
    return _tc_final(acc, hs, out_w, out_b)


def kernel(x, edge_index, node_time, edge_time, time_w, time_b, lin_w, lin_b,
           key_w, key_b, query_w, query_b, value_w, value_b, edge_w,
           skip_w, skip_b, out_w, out_b):
    return _pipeline(x, edge_index, node_time, edge_time, time_w, time_b,
                     lin_w, lin_b, key_w, key_b, query_w, query_b, value_w,
                     value_b, edge_w, skip_w, skip_b, out_w, out_b)


# speedup vs baseline: 14.9342x; 1.0017x over previous
import jax
import jax.numpy as jnp
from jax import lax
from jax.experimental import pallas as pl
from jax.experimental.pallas import tpu as pltpu
from jax.experimental.pallas import tpu_sc as plsc

N = 10000
E = 320000
D_IN = 128
HC = 32
NCLS = 2

NC = 2
NS = 16
NW = NC * NS
L = 16

B = 128
BLKS = -(-E // (NW * B))
EPW = BLKS * B
E_PAD = NW * EPW

ROWS_PER_TILE = 640
NP = NS * ROWS_PER_TILE
QROWS = N + L


def _tc_proj_body(x_ref, lwT, lb, qwT, qb, kwT, kb, vwT, vb, swT, sb,
                  qn, kn, vn, hs):
    xb = x_ref[...]
    h1 = jnp.maximum(jnp.dot(xb, lwT[...], preferred_element_type=jnp.float32)
                     + lb[...], 0.0)
    qn[...] = jnp.dot(h1, qwT[...], preferred_element_type=jnp.float32) + qb[...]
    kn[...] = jnp.dot(h1, kwT[...], preferred_element_type=jnp.float32) + kb[...]
    vn[...] = jnp.dot(h1, vwT[...], preferred_element_type=jnp.float32) + vb[...]
    hs[...] = jnp.dot(h1, swT[...], preferred_element_type=jnp.float32) + sb[...]


def _tc_proj(x, lin_w, lin_b, key_w, key_b, query_w, query_b, value_w, value_b,
             skip_w, skip_b):
    R = 2000
    grid = (N // R,)
    row_spec = pl.BlockSpec((R, D_IN), lambda i: (i, 0))
    out_spec = pl.BlockSpec((R, HC), lambda i: (i, 0))
    full = lambda shape: pl.BlockSpec(shape, lambda i: (0, 0))
    return pl.pallas_call(
        _tc_proj_body,
        grid=grid,
        in_specs=[row_spec,
                  full((D_IN, HC)), full((1, HC)),
                  full((HC, HC)), full((1, HC)),
                  full((HC, HC)), full((1, HC)),
                  full((HC, HC)), full((1, HC)),
                  full((HC, HC)), full((1, HC))],
        out_specs=[out_spec] * 4,
        out_shape=[jax.ShapeDtypeStruct((N, HC), jnp.float32)] * 4,
    )(x, lin_w.T, lin_b.reshape(1, HC),
      query_w.T, query_b.reshape(1, HC),
      key_w.T, key_b.reshape(1, HC),
      value_w.T, value_b.reshape(1, HC),
      skip_w.T, skip_b.reshape(1, HC))


def _sc_relt_body(nt_hbm, src_hbm, et_hbm, rel_hbm, nt_v, src_v, et_v, rel_v):
    cid = lax.axis_index("c")
    sid = lax.axis_index("s")
    wid = cid * NS + sid
    wbase = wid * EPW
    pltpu.sync_copy(nt_hbm, nt_v)
    pltpu.sync_copy(src_hbm.at[pl.ds(wbase, EPW)], src_v)
    pltpu.sync_copy(et_hbm.at[pl.ds(wbase, EPW)], et_v)

    def grp_body(g, _):
        off = g * L
        sidx = src_v[pl.ds(off, L)]
        r = plsc.load_gather(nt_v, [sidx]) - et_v[pl.ds(off, L)]
        rel_v[pl.ds(off, L)] = r
        return 0

    lax.fori_loop(0, EPW // L, grp_body, 0)
    pltpu.sync_copy(rel_v, rel_hbm.at[pl.ds(wbase, EPW)])


def _sc_relt(node_time, src_pad, et_pad):
    mesh = plsc.VectorSubcoreMesh(core_axis_name="c", subcore_axis_name="s")
    return pl.kernel(
        _sc_relt_body,
        out_type=jax.ShapeDtypeStruct((E_PAD,), jnp.float32),
        mesh=mesh,
        scratch_types=[
            pltpu.VMEM((N,), jnp.float32),
            pltpu.VMEM((EPW,), jnp.int32),
            pltpu.VMEM((EPW,), jnp.float32),
            pltpu.VMEM((EPW,), jnp.float32),
        ],
        compiler_params=pltpu.CompilerParams(needs_layout_passes=False, use_tc_tiling_on_sc=False),
    )(node_time, src_pad, et_pad)


def _tc_emat_body(rel_ref, twr, tbr, ewT, out_ref):
    r = rel_ref[0]
    enc = jnp.cos(r * twr[...] + tbr[...])
    out_ref[0] = jnp.dot(enc, ewT[...], preferred_element_type=jnp.float32)


def _tc_emat(rel_pad, time_w, time_b, edge_w):
    RB = 1024
    grid = (E_PAD // RB,)
    rel3 = rel_pad.reshape(E_PAD // RB, RB, 1)
    out = pl.pallas_call(
        _tc_emat_body,
        grid=grid,
        in_specs=[pl.BlockSpec((1, RB, 1), lambda i: (i, 0, 0)),
                  pl.BlockSpec((1, HC), lambda i: (0, 0)),
                  pl.BlockSpec((1, HC), lambda i: (0, 0)),
                  pl.BlockSpec((HC, HC), lambda i: (0, 0))],
        out_specs=pl.BlockSpec((1, RB, HC), lambda i: (i, 0, 0)),
        out_shape=jax.ShapeDtypeStruct((E_PAD // RB, RB, HC), jnp.float32),
    )(rel3, time_w.reshape(1, HC), time_b.reshape(1, HC), edge_w.T)
    return out.reshape(E_PAD, HC)


def _sc_attn_body(qn_hbm, kn_hbm, vn_hbm, em_hbm, src_hbm, dst_hbm, acc_hbm,
                  src_all, dst_all, qA, kA, vA, eA, qB, kB, vB, eB,
                  msg, acc_sh, semA, semB):
    cid = lax.axis_index("c")
    sid = lax.axis_index("s")
    wid = cid * NS + sid
    iota = lax.iota(jnp.int32, L)
    zf = jnp.zeros((L,), jnp.float32)

    pltpu.sync_copy(src_hbm.at[wid], src_all)
    pltpu.sync_copy(dst_hbm.at[wid], dst_all)

    def mrow(i, _):
        for j in range(3):
            msg[i, pl.ds(j * L, L)] = zf
        return 0
    lax.fori_loop(0, B, mrow, 0)
    tbase = sid * ROWS_PER_TILE
    for j in range(ROWS_PER_TILE // B):
        pltpu.sync_copy(msg, acc_sh.at[pl.ds(tbase + j * B, B)])
    plsc.subcore_barrier()

    def issue(blk, bufs, sem):
        qr, kr, vr, er = bufs
        idx = src_all.at[blk]
        pltpu.async_copy(kn_hbm.at[idx], kr, sem)
        pltpu.async_copy(vn_hbm.at[idx], vr, sem)
        pltpu.async_copy(qn_hbm.at[dst_all.at[blk]], qr, sem)
        pltpu.async_copy(em_hbm.at[pl.ds((wid * BLKS + blk) * B, B)], er, sem)

    def drain(blk, bufs, sem):
        qr, kr, vr, er = bufs
        idx = src_all.at[blk]
        pltpu.make_async_copy(kn_hbm.at[idx], kr, sem).wait()
        pltpu.make_async_copy(vn_hbm.at[idx], vr, sem).wait()
        pltpu.make_async_copy(qn_hbm.at[dst_all.at[blk]], qr, sem).wait()
        pltpu.make_async_copy(em_hbm.at[pl.ds((wid * BLKS + blk) * B, B)], er,
                              sem).wait()

    def compute(blk, bufs):
        qr, kr, vr, er = bufs

        def grp_body(g, _):
            ids = g * L + iota
            a0 = zf
            a1 = zf
            for c in range(HC):
                cv = jnp.full((L,), c, jnp.int32)
                qc = plsc.load_gather(qr, [ids, cv])
                kc = plsc.load_gather(kr, [ids, cv])
                ec = plsc.load_gather(er, [ids, cv])
                t = qc * (kc + ec)
                if c < 16:
                    a0 = a0 + t
                else:
                    a1 = a1 + t
            p0 = jnp.exp(a0 * 0.25)
            p1 = jnp.exp(a1 * 0.25)
            for c in range(HC):
                cv = jnp.full((L,), c, jnp.int32)
                vc = plsc.load_gather(vr, [ids, cv])
                ec = plsc.load_gather(er, [ids, cv])
                p = p0 if c < 16 else p1
                plsc.store_scatter(msg, [ids, cv], p * (vc + ec))
            plsc.store_scatter(msg, [ids, jnp.full((L,), 32, jnp.int32)], p0)
            plsc.store_scatter(msg, [ids, jnp.full((L,), 33, jnp.int32)], p1)
            return 0

        def grp2(h, _):
            grp_body(2 * h, 0)
            grp_body(2 * h + 1, 0)
            return 0

        lax.fori_loop(0, B // L // 2, grp2, 0)
        pltpu.sync_copy(msg, acc_sh.at[dst_all.at[blk]], add=True)

    bufsA = (qA, kA, vA, eA)
    bufsB = (qB, kB, vB, eB)
    issue(0, bufsA, semA)

    def pair_body(i, _):
        issue(2 * i + 1, bufsB, semB)
        drain(2 * i, bufsA, semA)
        compute(2 * i, bufsA)
        issue(2 * i + 2, bufsA, semA)
        drain(2 * i + 1, bufsB, semB)
        compute(2 * i + 1, bufsB)
        return 0

    lax.fori_loop(0, (BLKS - 1) // 2, pair_body, 0)
    drain(BLKS - 1, bufsA, semA)
    compute(BLKS - 1, bufsA)

    plsc.subcore_barrier()
    pltpu.sync_copy(acc_sh.at[pl.ds(tbase, ROWS_PER_TILE)],
                    acc_hbm.at[cid, pl.ds(tbase, ROWS_PER_TILE)])


def _sc_attn(qn_pad, kn, vn, e_mat, src_pad, dst_pad):
    mesh = plsc.VectorSubcoreMesh(core_axis_name="c", subcore_axis_name="s")
    rows = lambda: pltpu.VMEM((B, HC), jnp.float32)
    return pl.kernel(
        _sc_attn_body,
        out_type=jax.ShapeDtypeStruct((NC, NP, 48), jnp.float32),
        mesh=mesh,
        scratch_types=[
            pltpu.VMEM((BLKS, B), jnp.int32),
            pltpu.VMEM((BLKS, B), jnp.int32),
            rows(), rows(), rows(), rows(),
            rows(), rows(), rows(), rows(),
            pltpu.VMEM((B, 48), jnp.float32),
            pltpu.VMEM_SHARED((NP, 48), jnp.float32),
            pltpu.SemaphoreType.DMA,
            pltpu.SemaphoreType.DMA,
        ],
        compiler_params=pltpu.CompilerParams(needs_layout_passes=False, use_tc_tiling_on_sc=False),
    )(qn_pad, kn, vn, e_mat,
      src_pad.reshape(NW, BLKS, B), dst_pad.reshape(NW, BLKS, B))


def _tc_final_body(a0_ref, a1_ref, hs_ref, owT, ob, out_ref):
    a = a0_ref[...] + a1_ref[...]
    d0 = a[:, 32:33] + 1e-16
    d1 = a[:, 33:34] + 1e-16
    h0 = a[:, 0:16] / d0
    h1 = a[:, 16:32] / d1
    w = owT[...]
    o = (jnp.dot(h0, w[0:16, :], preferred_element_type=jnp.float32)
         + jnp.dot(h1, w[16:32, :], preferred_element_type=jnp.float32)
         + jnp.dot(hs_ref[...], w, preferred_element_type=jnp.float32)
         + ob[...])
    m = jnp.max(o, axis=1, keepdims=True)
    s = o - m
    out_ref[...] = s - jnp.log(jnp.sum(jnp.exp(s), axis=1, keepdims=True))


def _tc_final(acc, hs, out_w, out_b):
    R = 2000
    grid = (N // R,)
    return pl.pallas_call(
        _tc_final_body,
        grid=grid,
        in_specs=[pl.BlockSpec((R, 48), lambda i: (i, 0)),
                  pl.BlockSpec((R, 48), lambda i: (i, 0)),
                  pl.BlockSpec((R, HC), lambda i: (i, 0)),
                  pl.BlockSpec((HC, NCLS), lambda i: (0, 0)),
                  pl.BlockSpec((1, NCLS), lambda i: (0, 0))],
        out_specs=pl.BlockSpec((R, NCLS), lambda i: (i, 0)),
        out_shape=jax.ShapeDtypeStruct((N, NCLS), jnp.float32),
    )(acc[0, :N], acc[1, :N], hs, out_w.T, out_b.reshape(1, NCLS))


def _pipeline(x, edge_index, node_time, edge_time, time_w, time_b, lin_w,
              lin_b, key_w, key_b, query_w, query_b, value_w, value_b,
              edge_w, skip_w, skip_b, out_w, out_b):
    src = edge_index[0]
    dst = edge_index[1]
    pad = E_PAD - E
    src_pad = jnp.concatenate([src, jnp.zeros((pad,), jnp.int32)])
    dst_pad = jnp.concatenate([dst, jnp.full((pad,), N, jnp.int32)])
    et_pad = jnp.concatenate([edge_time.reshape(E), jnp.zeros((pad,), jnp.float32)])

    qn, kn, vn, hs = _tc_proj(x, lin_w, lin_b, key_w, key_b, query_w, query_b,
                              value_w, value_b, skip_w, skip_b)
    qn_pad = jnp.concatenate([qn, jnp.zeros((QROWS - N, HC), jnp.float32)])

    rel_pad = _sc_relt(node_time, src_pad, et_pad)
    e_mat = _tc_emat(rel_pad, time_w, time_b, edge_w)
    acc = _sc_attn(qn_pad, kn, vn, e_mat, src_pad, dst_pad)
    return _tc_final(acc, hs, out_w, out_b)


def kernel(x, edge_index, node_time, edge_time, time_w, time_b, lin_w, lin_b,
           key_w, key_b, query_w, query_b, value_w, value_b, edge_w,
           skip_w, skip_b, out_w, out_b):
    return _pipeline(x, edge_index, node_time, edge_time, time_w, time_b,
                     lin_w, lin_b, key_w, key_b, query_w, query_b, value_w,
                     value_b, edge_w, skip_w, skip_b, out_w, out_b)
